# baseline (speedup 1.0000x reference)
import functools

import jax
import jax.numpy as jnp
from jax import lax
from jax.experimental import pallas as pl
from jax.experimental.pallas import tpu as pltpu
from jax.experimental.pallas import tpu_sc as plsc


def kernel(inputs, neighbor_indices):
    B, L, T, C = inputs.shape
    _, K = neighbor_indices.shape
    D = T * C
    BL = B * L

    info = plsc.get_sparse_core_info()
    NC, NS = info.num_cores, info.num_subcores
    NW = NC * NS
    CH = 32
    NB = 3
    ZROW = BL
    LT = L // 128

    half = B // 2
    table_a = inputs[:half].reshape(half * L, 8, D // 8)
    table_b = inputs[half:].reshape(half * L, 8, D // 8)
    nidx_flat = neighbor_indices.reshape(L * K)
    maskT = (neighbor_indices != -1).astype(inputs.dtype).T.reshape(K, 1, L)

    mesh = plsc.VectorSubcoreMesh(core_axis_name="c", subcore_axis_name="s")

    def make_gather(nb, b0):
        R = nb * L * K
        RPW = R // NW
        WPB = NW // nb
        NCH = RPW // CH

        @functools.partial(
            pl.kernel,
            mesh=mesh,
            out_type=jax.ShapeDtypeStruct((R, 8, D // 8), inputs.dtype),
            scratch_types=[
                pltpu.VMEM((RPW,), jnp.int32),
                pltpu.VMEM((RPW,), jnp.int32),
                pltpu.VMEM((CH, 8, D // 8), jnp.float32),
                pltpu.VMEM((CH, 8, D // 8), jnp.float32),
                pltpu.VMEM((CH, 8, D // 8), jnp.float32),
                pltpu.SemaphoreType.DMA,
                pltpu.SemaphoreType.DMA,
                pltpu.SemaphoreType.DMA,
                pltpu.SemaphoreType.DMA,
                pltpu.SemaphoreType.DMA,
                pltpu.SemaphoreType.DMA,
            ],
        )
        def gather_k(table_h, nidx_h, out_h, raw_v, gidx_v, b0v, b1v, b2v,
                     gs0, gs1, gs2, ws0, ws1, ws2):
            wid = lax.axis_index("s") * NC + lax.axis_index("c")
            b = b0 + wid // WPB
            base = wid * RPW
            nbase = (wid % WPB) * RPW
            pltpu.sync_copy(nidx_h.at[pl.ds(nbase, RPW)], raw_v)
            bL = b * L
            for i in range(RPW // 16):
                v = raw_v[pl.ds(i * 16, 16)]
                gidx_v[pl.ds(i * 16, 16)] = jnp.maximum(v, 0) + bL

            bufs = (b0v, b1v, b2v)
            gsems = (gs0, gs1, gs2)
            wsems = (ws0, ws1, ws2)
            gh = [None] * NB
            wh = [None] * NB
            for p in range(NB - 1):
                gh[p] = pltpu.async_copy(
                    table_h.at[gidx_v.at[pl.ds(p * CH, CH)]], bufs[p], gsems[p])
            for c in range(NCH):
                j = c % NB
                gh[j].wait()
                wh[j] = pltpu.async_copy(
                    bufs[j], out_h.at[pl.ds(base + c * CH, CH)], wsems[j])
                n = c + NB - 1
                if n < NCH:
                    jn = n % NB
                    if wh[jn] is not None:
                        wh[jn].wait()
                    gh[jn] = pltpu.async_copy(
                        table_h.at[gidx_v.at[pl.ds(n * CH, CH)]],
                        bufs[jn], gsems[jn])
            for j in range(NB):
                if wh[j] is not None:
                    wh[j].wait()

        return gather_k

    def repack_body(x_ref, m_ref, y_ref):
        xb = x_ref[0, :, 0]
        for s in range(8):
            for lt in range(LT):
                mrow = m_ref[0, 0, 128 * lt:128 * (lt + 1)]
                tr = xb[128 * lt:128 * (lt + 1), s, :].T * mrow[None, :]
                y_ref[0, 0, 2 * s, :, 128 * lt:128 * (lt + 1)] = tr[:64]
                y_ref[0, 0, 2 * s + 1, :, 128 * lt:128 * (lt + 1)] = tr[64:]

    def repack_first(x, nb):
        x5 = x.reshape(nb, L, K, 8, D // 8)
        return pl.pallas_call(
            repack_body,
            grid=(nb, K),
            in_specs=[
                pl.BlockSpec(
                    (1, L, 1, 8, D // 8), lambda b, k: (b, 0, k, 0, 0)),
                pl.BlockSpec((1, 1, L), lambda b, k: (k, 0, 0)),
            ],
            out_specs=pl.BlockSpec(
                (1, 1, T, C, L), lambda b, k: (b, k, 0, 0, 0)),
            out_shape=jax.ShapeDtypeStruct((B, K, T, C, L), inputs.dtype),
        )(x5, maskT)

    def repack_rest(x, nb, b0, y_prev):
        x5 = x.reshape(nb, L, K, 8, D // 8)

        def body(x_ref, m_ref, y_in_ref, y_ref):
            repack_body(x_ref, m_ref, y_ref)

        return pl.pallas_call(
            body,
            grid=(nb, K),
            in_specs=[
                pl.BlockSpec(
                    (1, L, 1, 8, D // 8), lambda b, k: (b, 0, k, 0, 0)),
                pl.BlockSpec((1, 1, L), lambda b, k: (k, 0, 0)),
                pl.BlockSpec(memory_space=pl.ANY),
            ],
            out_specs=pl.BlockSpec(
                (1, 1, T, C, L), lambda b, k: (b0 + b, k, 0, 0, 0)),
            out_shape=jax.ShapeDtypeStruct((B, K, T, C, L), inputs.dtype),
            input_output_aliases={2: 0},
        )(x5, maskT, y_prev)

    x_a = make_gather(half, 0)(table_a, nidx_flat)
    x_b = make_gather(half, 0)(table_b, nidx_flat)
    y_a = repack_first(x_a, half)
    y = repack_rest(x_b, half, half, y_a)
    return y.transpose(0, 4, 1, 2, 3)


# speedup vs baseline: 1.0483x; 1.0483x over previous
import functools

import jax
import jax.numpy as jnp
from jax import lax
from jax.experimental import pallas as pl
from jax.experimental.pallas import tpu as pltpu
from jax.experimental.pallas import tpu_sc as plsc


def kernel(inputs, neighbor_indices):
    B, L, T, C = inputs.shape
    _, K = neighbor_indices.shape
    D = T * C
    BL = B * L

    info = plsc.get_sparse_core_info()
    NC, NS = info.num_cores, info.num_subcores
    NW = NC * NS
    CH = 32
    NB = 3
    LT = L // 128

    table = inputs.reshape(BL, 8, D // 8)
    nidx_flat = neighbor_indices.reshape(L * K)
    maskT = (neighbor_indices != -1).astype(inputs.dtype).T.reshape(K, 1, L)

    mesh = plsc.VectorSubcoreMesh(core_axis_name="c", subcore_axis_name="s")

    def make_gather(nb, b0):
        R = nb * L * K
        RPW = R // NW
        WPB = NW // nb
        NCH = RPW // CH

        @functools.partial(
            pl.kernel,
            mesh=mesh,
            out_type=jax.ShapeDtypeStruct((R, 8, D // 8), inputs.dtype),
            scratch_types=[
                pltpu.VMEM((RPW,), jnp.int32),
                pltpu.VMEM((RPW,), jnp.int32),
                pltpu.VMEM((CH, 8, D // 8), jnp.float32),
                pltpu.VMEM((CH, 8, D // 8), jnp.float32),
                pltpu.VMEM((CH, 8, D // 8), jnp.float32),
                pltpu.SemaphoreType.DMA,
                pltpu.SemaphoreType.DMA,
                pltpu.SemaphoreType.DMA,
                pltpu.SemaphoreType.DMA,
                pltpu.SemaphoreType.DMA,
                pltpu.SemaphoreType.DMA,
            ],
        )
        def gather_k(table_h, nidx_h, out_h, raw_v, gidx_v, b0v, b1v, b2v,
                     gs0, gs1, gs2, ws0, ws1, ws2):
            wid = lax.axis_index("s") * NC + lax.axis_index("c")
            b = b0 + wid // WPB
            base = wid * RPW
            nbase = (wid % WPB) * RPW
            pltpu.sync_copy(nidx_h.at[pl.ds(nbase, RPW)], raw_v)
            bL = b * L
            for i in range(RPW // 16):
                v = raw_v[pl.ds(i * 16, 16)]
                gidx_v[pl.ds(i * 16, 16)] = jnp.maximum(v, 0) + bL

            bufs = (b0v, b1v, b2v)
            gsems = (gs0, gs1, gs2)
            wsems = (ws0, ws1, ws2)
            gh = [None] * NB
            wh = [None] * NB
            for p in range(NB - 1):
                gh[p] = pltpu.async_copy(
                    table_h.at[gidx_v.at[pl.ds(p * CH, CH)]], bufs[p], gsems[p])
            for c in range(NCH):
                j = c % NB
                gh[j].wait()
                wh[j] = pltpu.async_copy(
                    bufs[j], out_h.at[pl.ds(base + c * CH, CH)], wsems[j])
                n = c + NB - 1
                if n < NCH:
                    jn = n % NB
                    if wh[jn] is not None:
                        wh[jn].wait()
                    gh[jn] = pltpu.async_copy(
                        table_h.at[gidx_v.at[pl.ds(n * CH, CH)]],
                        bufs[jn], gsems[jn])
            for j in range(NB):
                if wh[j] is not None:
                    wh[j].wait()

        return gather_k

    def repack_body(x_ref, m_ref, y_ref):
        xb = x_ref[0, :, 0]
        for s in range(8):
            for lt in range(LT):
                mrow = m_ref[0, 0, 128 * lt:128 * (lt + 1)]
                tr = xb[128 * lt:128 * (lt + 1), s, :].T * mrow[None, :]
                y_ref[0, 0, 2 * s, :, 128 * lt:128 * (lt + 1)] = tr[:64]
                y_ref[0, 0, 2 * s + 1, :, 128 * lt:128 * (lt + 1)] = tr[64:]

    def repack_first(x, nb):
        x5 = x.reshape(nb, L, K, 8, D // 8)
        return pl.pallas_call(
            repack_body,
            grid=(nb, K),
            in_specs=[
                pl.BlockSpec(
                    (1, L, 1, 8, D // 8), lambda b, k: (b, 0, k, 0, 0)),
                pl.BlockSpec((1, 1, L), lambda b, k: (k, 0, 0)),
            ],
            out_specs=pl.BlockSpec(
                (1, 1, T, C, L), lambda b, k: (b, k, 0, 0, 0)),
            out_shape=jax.ShapeDtypeStruct((B, K, T, C, L), inputs.dtype),
        )(x5, maskT)

    def repack_rest(x, nb, b0, y_prev):
        x5 = x.reshape(nb, L, K, 8, D // 8)

        def body(x_ref, m_ref, y_in_ref, y_ref):
            repack_body(x_ref, m_ref, y_ref)

        return pl.pallas_call(
            body,
            grid=(nb, K),
            in_specs=[
                pl.BlockSpec(
                    (1, L, 1, 8, D // 8), lambda b, k: (b, 0, k, 0, 0)),
                pl.BlockSpec((1, 1, L), lambda b, k: (k, 0, 0)),
                pl.BlockSpec(memory_space=pl.ANY),
            ],
            out_specs=pl.BlockSpec(
                (1, 1, T, C, L), lambda b, k: (b0 + b, k, 0, 0, 0)),
            out_shape=jax.ShapeDtypeStruct((B, K, T, C, L), inputs.dtype),
            input_output_aliases={2: 0},
        )(x5, maskT, y_prev)

    half = B // 2
    x_a = make_gather(half, 0)(table, nidx_flat)
    x_b = make_gather(half, half)(table, nidx_flat)
    y_a = repack_first(x_a, half)
    y = repack_rest(x_b, half, half, y_a)
    return y.transpose(0, 4, 1, 2, 3)
